# baseline (speedup 1.0000x reference)
import functools

import jax
import jax.numpy as jnp
from jax import lax
from jax.experimental import pallas as pl
from jax.experimental.pallas import tpu as pltpu
from jax.experimental.pallas import tpu_sc as plsc

_B = 16384
_D = 64
_V = 100000

_info = plsc.get_sparse_core_info()
_NC = _info.num_cores
_NS = _info.num_subcores
_NW = _NC * _NS
_L = 16

_TPW = 25
_SLAB_T = 13
_SLAB_C = _SLAB_T * 128
_MAX_T0 = 768
_TAIL0 = 99968
_CHUNK = 1024
_NCHUNK = _B // _CHUNK

_mesh = plsc.VectorSubcoreMesh(core_axis_name="c", subcore_axis_name="s")


@functools.partial(
    pl.kernel,
    mesh=_mesh,
    out_type=jax.ShapeDtypeStruct((_B + 1, 2 * _D), jnp.float32),
    scratch_types=[
        pltpu.VMEM((_D, _SLAB_C), jnp.float32),
        pltpu.VMEM((_CHUNK,), jnp.int32),
        pltpu.VMEM((_CHUNK + _L,), jnp.int32),
        pltpu.VMEM((_CHUNK + _L,), jnp.int32),
        pltpu.VMEM(((_CHUNK + _L) // _L, _L), jnp.int32),
        pltpu.VMEM((2, _L, 2 * _D), jnp.float32),
        pltpu.SemaphoreType.DMA,
        pltpu.SemaphoreType.DMA,
        pltpu.SemaphoreType.DMA,
    ],
    compiler_params=pltpu.CompilerParams(
        use_tc_tiling_on_sc=True, needs_layout_passes=False
    ),
)
def _sc_gather(tt_hbm, idx_hbm, tail_hbm, out_hbm, slab_v, idxc_v, clist_v,
               blist_v, b2d_v, rows_v, slab_sem, idx_sem, sc_sem):
    wid = lax.axis_index("s") * _NC + lax.axis_index("c")
    lanes = lax.iota(jnp.int32, _L)
    junk_b = jnp.full((_L,), _B, jnp.int32)

    def slab_descs(start_col):
        return [
            pltpu.make_async_copy(
                tt_hbm.at[pl.ds(8 * s, 8), pl.ds(start_col, _SLAB_C)],
                slab_v.at[pl.ds(8 * s, 8), :],
                slab_sem,
            )
            for s in range(8)
        ]

    def run_pass(lo, hi, staged0, slab, wait_slab, swap_idx=False):
        def chunk_body(ch, carry):
            pltpu.async_copy(
                idx_hbm.at[pl.ds(ch * _CHUNK, _CHUNK)], idxc_v, idx_sem
            ).wait()
            lo_v = jnp.full((_L,), lo, jnp.int32)
            hi_v = jnp.full((_L,), hi, jnp.int32)
            st_v = jnp.full((_L,), staged0, jnp.int32)

            @plsc.parallel_loop(0, _CHUNK // _L, carry=jnp.int32(0))
            def cnt(g, cnt):
                iv = idxc_v[pl.ds(g * _L, _L)]
                m = (iv >= lo_v) & (iv < hi_v)
                n = plsc.all_reduce_population_count(m)[0]
                plsc.store_compressed(
                    clist_v.at[pl.ds(cnt, _L)], iv - st_v, mask=m
                )
                bv = ch * _CHUNK + g * _L + lanes
                plsc.store_compressed(blist_v.at[pl.ds(cnt, _L)], bv, mask=m)
                return cnt + n
            clist_v[pl.ds(cnt, _L)] = jnp.zeros((_L,), jnp.int32)
            blist_v[pl.ds(cnt, _L)] = junk_b
            ng = (cnt + _L - 1) // _L

            @plsc.parallel_loop(0, ng)
            def _(g):
                bv = blist_v[pl.ds(g * _L, _L)]
                plsc.store_scatter(b2d_v, [jnp.full((_L,), g, jnp.int32), lanes], bv)

            if wait_slab:
                @pl.when(ch == 0)
                def _():
                    for dsc in slab_descs(staged0):
                        dsc.wait()

            def group(g, c2):
                r = g & 1

                @pl.when(g >= 2)
                def _():
                    pltpu.make_async_copy(
                        rows_v.at[(g - 2) & 1], out_hbm.at[b2d_v.at[g - 2]], sc_sem
                    ).wait()

                cv = clist_v[pl.ds(g * _L, _L)]
                r_v = jnp.full((_L,), r, jnp.int32)

                @plsc.parallel_loop(0, _D, unroll=8)
                def _(d):
                    d_v = jnp.full((_L,), d, jnp.int32)
                    gi = [cv, d_v] if swap_idx else [d_v, cv]
                    vals = plsc.load_gather(slab, gi)
                    plsc.store_scatter(rows_v, [r_v, lanes, d_v], vals)

                pltpu.async_copy(rows_v.at[r], out_hbm.at[b2d_v.at[g]], sc_sem)
                return c2

            lax.fori_loop(0, ng, group, 0)

            def drain(g, c2):
                pltpu.make_async_copy(
                    rows_v.at[g & 1], out_hbm.at[b2d_v.at[g]], sc_sem
                ).wait()
                return c2

            lax.fori_loop(lax.max(ng - 2, 0), ng, drain, 0)
            return carry

        lax.fori_loop(0, _NCHUNK, chunk_body, 0)

    t0 = wid * _TPW
    sa = 128 * lax.min(t0, _MAX_T0)
    sb = 128 * lax.min(t0 + _SLAB_T, _MAX_T0)
    a_lo = 128 * t0
    a_hi = lax.min(a_lo + _SLAB_C, _TAIL0)
    b_lo = a_lo + _SLAB_C
    b_hi = lax.min(128 * (t0 + _TPW), _TAIL0)

    for dsc in slab_descs(sa):
        dsc.start()
    run_pass(a_lo, a_hi, sa, slab_v, True)

    for dsc in slab_descs(sb):
        dsc.start()
    run_pass(b_lo, b_hi, sb, slab_v, True)

    @pl.when(wid == _NW - 1)
    def _():
        pltpu.async_copy(
            tail_hbm, slab_v.at[pl.ds(0, 32), pl.ds(0, 128)], slab_sem
        ).wait()
        run_pass(_TAIL0, _V, _TAIL0, slab_v, False, swap_idx=True)


def kernel(hyperparameters, embedding_table):
    idx = jnp.squeeze(hyperparameters, axis=1)
    tt = embedding_table.T
    tail_tab = jnp.pad(embedding_table[_TAIL0:, :], ((0, 0), (0, _D)))
    wide = _sc_gather(tt, idx, tail_tab)
    return wide[:_B, :_D]


# speedup vs baseline: 3.4716x; 3.4716x over previous
import functools

import jax
import jax.numpy as jnp
from jax import lax
from jax.experimental import pallas as pl
from jax.experimental.pallas import tpu as pltpu
from jax.experimental.pallas import tpu_sc as plsc

_B = 16384
_D = 64

_info = plsc.get_sparse_core_info()
_NC = _info.num_cores
_NS = _info.num_subcores
_NW = _NC * _NS
_B_PER_W = _B // _NW

_mesh = plsc.VectorSubcoreMesh(core_axis_name="c", subcore_axis_name="s")


@functools.partial(
    pl.kernel,
    mesh=_mesh,
    out_type=jax.ShapeDtypeStruct((_B, _D), jnp.float32),
    scratch_types=[
        pltpu.VMEM((_B_PER_W,), jnp.int32),
        pltpu.VMEM((_B_PER_W, _D), jnp.float32),
        pltpu.SemaphoreType.DMA,
    ],
    compiler_params=pltpu.CompilerParams(use_tc_tiling_on_sc=False),
)
def _sc_gather(table_hbm, idx_hbm, out_hbm, idx_v, rows_v, sem):
    wid = lax.axis_index("s") * _NC + lax.axis_index("c")
    base = wid * _B_PER_W
    pltpu.sync_copy(idx_hbm.at[pl.ds(base, _B_PER_W)], idx_v)
    pltpu.async_copy(table_hbm.at[idx_v], rows_v, sem).wait()
    pltpu.sync_copy(rows_v, out_hbm.at[pl.ds(base, _B_PER_W)])


def kernel(hyperparameters, embedding_table):
    idx = jnp.squeeze(hyperparameters, axis=1)
    return _sc_gather(embedding_table, idx)
